# baseline (speedup 1.0000x reference)
import functools

import jax
import jax.numpy as jnp
from jax import lax
from jax.experimental import pallas as pl
from jax.experimental.pallas import tpu as pltpu
from jax.experimental.pallas import tpu_sc as plsc

N = 10000
HID = 128

K = 128
NC = 2
NS = 16
NW = NC * NS
TOT_CHUNKS = 2560
E_PAD = TOT_CHUNKS * K
STAGE = 8
FAST_CORE = 1
CH_FAST = 20 * STAGE
CH_SLOW = 0 * STAGE
ACC_ROWS = 10112
R_SLICE = ACC_ROWS // NS

_PREC = jax.lax.Precision.DEFAULT


def _sc_aggregate(h, col_r, row_r, zer):
  mesh = plsc.VectorSubcoreMesh(core_axis_name="c", subcore_axis_name="s")

  @functools.partial(
      pl.kernel,
      mesh=mesh,
      out_type=jax.ShapeDtypeStruct((NC, ACC_ROWS, HID), jnp.float32),
      scratch_types=[
          pltpu.VMEM((STAGE, K), jnp.int32),
          pltpu.VMEM((STAGE, K), jnp.int32),
          pltpu.VMEM((K, HID), jnp.float32),
          pltpu.VMEM((K, HID), jnp.float32),
          pltpu.VMEM_SHARED((ACC_ROWS, HID), jnp.float32),
          pltpu.SemaphoreType.DMA,
          pltpu.SemaphoreType.DMA,
      ],
  )
  def k(h_hbm, col_hbm, row_hbm, zer_hbm, out_hbm, colv, rowv, buf_a, buf_b,
        acc, sem_a, sem_b):
    c = lax.axis_index("c")
    s = lax.axis_index("s")
    pltpu.sync_copy(zer_hbm.at[pl.ds(s * R_SLICE, R_SLICE)],
                    acc.at[pl.ds(s * R_SLICE, R_SLICE)])
    plsc.subcore_barrier()

    def run_stage(base):
      pltpu.sync_copy(col_hbm.at[pl.ds(base, STAGE)], colv)
      pltpu.sync_copy(row_hbm.at[pl.ds(base, STAGE)], rowv)

      @pl.loop(0, STAGE, step=2)
      def _(i):
        cp_a = pltpu.async_copy(h_hbm.at[colv.at[i]], buf_a, sem_a)
        cp_b = pltpu.async_copy(h_hbm.at[colv.at[i + 1]], buf_b, sem_b)
        cp_a.wait()
        pltpu.sync_copy(buf_a, acc.at[rowv.at[i]], add=True)
        cp_b.wait()
        pltpu.sync_copy(buf_b, acc.at[rowv.at[i + 1]], add=True)

    @pl.when(c == FAST_CORE)
    def _():
      for st in range(CH_FAST // STAGE):
        run_stage(s * CH_FAST + st * STAGE)

    @pl.when(c != FAST_CORE)
    def _():
      for st in range(CH_SLOW // STAGE):
        run_stage(NS * CH_FAST + s * CH_SLOW + st * STAGE)

    plsc.subcore_barrier()
    pltpu.sync_copy(acc.at[pl.ds(s * R_SLICE, R_SLICE)],
                    out_hbm.at[c, pl.ds(s * R_SLICE, R_SLICE)])

  return k(h, col_r, row_r, zer)


def _embed_body(x_ref, w_ref, b_ref, o_ref):
  o_ref[...] = jnp.dot(x_ref[...], w_ref[...], precision=_PREC) + b_ref[...]


def _embed(x2d, W, b2d):
  grid = (N // 1000,)
  return pl.pallas_call(
      _embed_body,
      grid=grid,
      in_specs=[
          pl.BlockSpec((1000, HID), lambda i: (i, 0)),
          pl.BlockSpec((HID, HID), lambda i: (0, 0)),
          pl.BlockSpec((1, HID), lambda i: (0, 0)),
      ],
      out_specs=pl.BlockSpec((1000, HID), lambda i: (i, 0)),
      out_shape=jax.ShapeDtypeStruct((N, HID), jnp.float32),
  )(x2d, W, b2d)


def _layer_body(p_ref, w_ref, b_ref, o_ref):
  aggr = p_ref[0] + p_ref[1]
  o_ref[...] = jax.nn.relu(
      jnp.dot(aggr, w_ref[...], precision=_PREC) + b_ref[...])


def _layer(partials, W, b2d):
  blk = ACC_ROWS // 8
  return pl.pallas_call(
      _layer_body,
      grid=(8,),
      in_specs=[
          pl.BlockSpec((NC, blk, HID), lambda i: (0, i, 0)),
          pl.BlockSpec((HID, HID), lambda i: (0, 0)),
          pl.BlockSpec((1, HID), lambda i: (0, 0)),
      ],
      out_specs=pl.BlockSpec((blk, HID), lambda i: (i, 0)),
      out_shape=jax.ShapeDtypeStruct((ACC_ROWS, HID), jnp.float32),
  )(partials, W, b2d)


def _final_body(p_ref, w2_ref, b2_ref, wc1_ref, bc1_ref, wc2_ref, bc2_ref,
                o_ref):
  aggr = p_ref[0] + p_ref[1]
  h = jax.nn.relu(jnp.dot(aggr, w2_ref[...], precision=_PREC) + b2_ref[...])
  hc = jax.nn.relu(jnp.dot(h, wc1_ref[...], precision=_PREC) + bc1_ref[...])
  o_ref[...] = jnp.dot(hc, wc2_ref[...], precision=_PREC) + bc2_ref[0, 0]


def _final(partials, W2, b2_2d, Wc1, bc1_2d, Wc2p, bc2_2d):
  blk = ACC_ROWS // 8
  return pl.pallas_call(
      _final_body,
      grid=(8,),
      in_specs=[
          pl.BlockSpec((NC, blk, HID), lambda i: (0, i, 0)),
          pl.BlockSpec((HID, HID), lambda i: (0, 0)),
          pl.BlockSpec((1, HID), lambda i: (0, 0)),
          pl.BlockSpec((HID, HID // 2), lambda i: (0, 0)),
          pl.BlockSpec((1, HID // 2), lambda i: (0, 0)),
          pl.BlockSpec((HID // 2, HID), lambda i: (0, 0)),
          pl.BlockSpec((1, 1), lambda i: (0, 0)),
      ],
      out_specs=pl.BlockSpec((blk, HID), lambda i: (i, 0)),
      out_shape=jax.ShapeDtypeStruct((ACC_ROWS, HID), jnp.float32),
  )(partials, W2, b2_2d, Wc1, bc1_2d, Wc2p, bc2_2d)


def kernel(x, edge_index, W_embed, b_embed, W1, b1, W2, b2, Wc1, bc1, Wc2,
           bc2):
  x2d = x.reshape(N, HID)
  row = edge_index[0].astype(jnp.int32)
  col = edge_index[1].astype(jnp.int32)
  npad = E_PAD - row.shape[0]
  row_r = jnp.concatenate(
      [row, jnp.full((npad,), N, jnp.int32)]).reshape(TOT_CHUNKS, K)
  col_r = jnp.concatenate(
      [col, jnp.zeros((npad,), jnp.int32)]).reshape(TOT_CHUNKS, K)
  zer = jnp.zeros((ACC_ROWS, HID), jnp.float32)

  b_embed2d = b_embed.reshape(1, HID)
  b1_2d = b1.reshape(1, HID)
  b2_2d = b2.reshape(1, HID)
  bc1_2d = bc1.reshape(1, HID // 2)
  bc2_2d = bc2.reshape(1, 1)
  Wc2p = jnp.zeros((HID // 2, HID), jnp.float32).at[:, 0].set(Wc2[:, 0])

  h = _embed(x2d, W_embed, b_embed2d)
  p1 = _sc_aggregate(h, col_r, row_r, zer)
  h1 = _layer(p1, W1, b1_2d)
  p2 = _sc_aggregate(h1, col_r, row_r, zer)
  out2d = _final(p2, W2, b2_2d, Wc1, bc1_2d, Wc2p, bc2_2d)
  return out2d[:N, 0]


# speedup vs baseline: 1.7104x; 1.7104x over previous
import functools

import jax
import jax.numpy as jnp
from jax import lax
from jax.experimental import pallas as pl
from jax.experimental.pallas import tpu as pltpu
from jax.experimental.pallas import tpu_sc as plsc

N = 10000
HID = 128

K = 128
NC = 2
NS = 16
NW = NC * NS
TOT_CHUNKS = 2560
E_PAD = TOT_CHUNKS * K
STAGE = 8
FAST_CORE = 1
CH_FAST = 19 * STAGE
CH_SLOW = 1 * STAGE
ACC_ROWS = 10112
R_SLICE = ACC_ROWS // NS

_PREC = jax.lax.Precision.DEFAULT


def _sc_aggregate(h, col_r, row_r, zer):
  mesh = plsc.VectorSubcoreMesh(core_axis_name="c", subcore_axis_name="s")

  @functools.partial(
      pl.kernel,
      mesh=mesh,
      out_type=jax.ShapeDtypeStruct((NC, ACC_ROWS, HID), jnp.float32),
      scratch_types=[
          pltpu.VMEM((STAGE, K), jnp.int32),
          pltpu.VMEM((STAGE, K), jnp.int32),
          pltpu.VMEM((K, HID), jnp.float32),
          pltpu.VMEM((K, HID), jnp.float32),
          pltpu.VMEM_SHARED((ACC_ROWS, HID), jnp.float32),
          pltpu.SemaphoreType.DMA,
          pltpu.SemaphoreType.DMA,
      ],
  )
  def k(h_hbm, col_hbm, row_hbm, zer_hbm, out_hbm, colv, rowv, buf_a, buf_b,
        acc, sem_a, sem_b):
    c = lax.axis_index("c")
    s = lax.axis_index("s")
    pltpu.sync_copy(zer_hbm.at[pl.ds(s * R_SLICE, R_SLICE)],
                    acc.at[pl.ds(s * R_SLICE, R_SLICE)])
    plsc.subcore_barrier()

    def run_stage(base):
      pltpu.sync_copy(col_hbm.at[pl.ds(base, STAGE)], colv)
      pltpu.sync_copy(row_hbm.at[pl.ds(base, STAGE)], rowv)

      @pl.loop(0, STAGE, step=2)
      def _(i):
        cp_a = pltpu.async_copy(h_hbm.at[colv.at[i]], buf_a, sem_a)
        cp_b = pltpu.async_copy(h_hbm.at[colv.at[i + 1]], buf_b, sem_b)
        cp_a.wait()
        pltpu.sync_copy(buf_a, acc.at[rowv.at[i]], add=True)
        cp_b.wait()
        pltpu.sync_copy(buf_b, acc.at[rowv.at[i + 1]], add=True)

    @pl.when(c == FAST_CORE)
    def _():
      for st in range(CH_FAST // STAGE):
        run_stage(s * CH_FAST + st * STAGE)

    @pl.when(c != FAST_CORE)
    def _():
      for st in range(CH_SLOW // STAGE):
        run_stage(NS * CH_FAST + s * CH_SLOW + st * STAGE)

    plsc.subcore_barrier()
    pltpu.sync_copy(acc.at[pl.ds(s * R_SLICE, R_SLICE)],
                    out_hbm.at[c, pl.ds(s * R_SLICE, R_SLICE)])

  return k(h, col_r, row_r, zer)


def _embed_body(x_ref, w_ref, b_ref, o_ref):
  o_ref[...] = jnp.dot(x_ref[...], w_ref[...], precision=_PREC) + b_ref[...]


def _embed(x2d, W, b2d):
  grid = (N // 1000,)
  return pl.pallas_call(
      _embed_body,
      grid=grid,
      in_specs=[
          pl.BlockSpec((1000, HID), lambda i: (i, 0)),
          pl.BlockSpec((HID, HID), lambda i: (0, 0)),
          pl.BlockSpec((1, HID), lambda i: (0, 0)),
      ],
      out_specs=pl.BlockSpec((1000, HID), lambda i: (i, 0)),
      out_shape=jax.ShapeDtypeStruct((N, HID), jnp.float32),
  )(x2d, W, b2d)


def _layer_body(p_ref, w_ref, b_ref, o_ref):
  aggr = p_ref[0] + p_ref[1]
  o_ref[...] = jax.nn.relu(
      jnp.dot(aggr, w_ref[...], precision=_PREC) + b_ref[...])


def _layer(partials, W, b2d):
  blk = ACC_ROWS // 8
  return pl.pallas_call(
      _layer_body,
      grid=(8,),
      in_specs=[
          pl.BlockSpec((NC, blk, HID), lambda i: (0, i, 0)),
          pl.BlockSpec((HID, HID), lambda i: (0, 0)),
          pl.BlockSpec((1, HID), lambda i: (0, 0)),
      ],
      out_specs=pl.BlockSpec((blk, HID), lambda i: (i, 0)),
      out_shape=jax.ShapeDtypeStruct((ACC_ROWS, HID), jnp.float32),
  )(partials, W, b2d)


def _final_body(p_ref, w2_ref, b2_ref, wc1_ref, bc1_ref, wc2_ref, bc2_ref,
                o_ref):
  aggr = p_ref[0] + p_ref[1]
  h = jax.nn.relu(jnp.dot(aggr, w2_ref[...], precision=_PREC) + b2_ref[...])
  hc = jax.nn.relu(jnp.dot(h, wc1_ref[...], precision=_PREC) + bc1_ref[...])
  o_ref[...] = jnp.dot(hc, wc2_ref[...], precision=_PREC) + bc2_ref[0, 0]


def _final(partials, W2, b2_2d, Wc1, bc1_2d, Wc2p, bc2_2d):
  blk = ACC_ROWS // 8
  return pl.pallas_call(
      _final_body,
      grid=(8,),
      in_specs=[
          pl.BlockSpec((NC, blk, HID), lambda i: (0, i, 0)),
          pl.BlockSpec((HID, HID), lambda i: (0, 0)),
          pl.BlockSpec((1, HID), lambda i: (0, 0)),
          pl.BlockSpec((HID, HID // 2), lambda i: (0, 0)),
          pl.BlockSpec((1, HID // 2), lambda i: (0, 0)),
          pl.BlockSpec((HID // 2, HID), lambda i: (0, 0)),
          pl.BlockSpec((1, 1), lambda i: (0, 0)),
      ],
      out_specs=pl.BlockSpec((blk, HID), lambda i: (i, 0)),
      out_shape=jax.ShapeDtypeStruct((ACC_ROWS, HID), jnp.float32),
  )(partials, W2, b2_2d, Wc1, bc1_2d, Wc2p, bc2_2d)


def kernel(x, edge_index, W_embed, b_embed, W1, b1, W2, b2, Wc1, bc1, Wc2,
           bc2):
  x2d = x.reshape(N, HID)
  row = edge_index[0].astype(jnp.int32)
  col = edge_index[1].astype(jnp.int32)
  npad = E_PAD - row.shape[0]
  row_r = jnp.concatenate(
      [row, jnp.full((npad,), N, jnp.int32)]).reshape(TOT_CHUNKS, K)
  col_r = jnp.concatenate(
      [col, jnp.zeros((npad,), jnp.int32)]).reshape(TOT_CHUNKS, K)
  zer = jnp.zeros((ACC_ROWS, HID), jnp.float32)

  b_embed2d = b_embed.reshape(1, HID)
  b1_2d = b1.reshape(1, HID)
  b2_2d = b2.reshape(1, HID)
  bc1_2d = bc1.reshape(1, HID // 2)
  bc2_2d = bc2.reshape(1, 1)
  Wc2p = jnp.zeros((HID // 2, HID), jnp.float32).at[:, 0].set(Wc2[:, 0])

  h = _embed(x2d, W_embed, b_embed2d)
  p1 = _sc_aggregate(h, col_r, row_r, zer)
  h1 = _layer(p1, W1, b1_2d)
  p2 = _sc_aggregate(h1, col_r, row_r, zer)
  out2d = _final(p2, W2, b2_2d, Wc1, bc1_2d, Wc2p, bc2_2d)
  return out2d[:N, 0]
